# baseline (speedup 1.0000x reference)
import functools

import jax
import jax.numpy as jnp
from jax import lax
from jax.experimental import pallas as pl
from jax.experimental.pallas import tpu as pltpu
from jax.experimental.pallas import tpu_sc as plsc

B, T, D, L = 16, 512, 1024, 2048
NW = 32
RPW = (B * L) // NW
CH = 16
NCH = RPW // CH
NBUF = 7


def _route_body(dur_ref, maxlen_ref, idx_ref, mel_ref, nvw_ref):
    dur = jnp.maximum(dur_ref[...], 0)
    rk = lax.broadcasted_iota(jnp.int32, (T, T), 0)
    ci = lax.broadcasted_iota(jnp.int32, (T, T), 1)
    tri = (rk <= ci).astype(jnp.float32)
    cum = jnp.dot(dur.astype(jnp.float32), tri,
                  preferred_element_type=jnp.float32)
    cumi = cum.astype(jnp.int32)
    ones8 = jnp.ones((8, T), jnp.float32)
    pos = lax.broadcasted_iota(jnp.int32, (T, L), 1)
    for b in range(B):
        cmp = (cumi[b, :][:, None] <= pos).astype(jnp.float32)
        cnt = jnp.dot(ones8, cmp, preferred_element_type=jnp.float32)
        idx = jnp.minimum(cnt[0:1, :].astype(jnp.int32), T - 1)
        idx_ref[b, :, :] = b * T + idx
        total = jnp.sum(dur[b, :])
        mel_ref[b] = total
        nvw_ref[b] = jnp.clip(jnp.minimum(total, maxlen_ref[0]), 0, L)


def _route(durations, max_len_arr):
    return pl.pallas_call(
        _route_body,
        in_specs=[
            pl.BlockSpec((B, T), lambda: (0, 0)),
            pl.BlockSpec(memory_space=pltpu.SMEM),
        ],
        out_specs=[
            pl.BlockSpec((B, 1, L), lambda: (0, 0, 0)),
            pl.BlockSpec(memory_space=pltpu.SMEM),
            pl.BlockSpec(memory_space=pltpu.SMEM),
        ],
        out_shape=[
            jax.ShapeDtypeStruct((B, 1, L), jnp.int32),
            jax.ShapeDtypeStruct((B,), jnp.int32),
            jax.ShapeDtypeStruct((B + 16,), jnp.int32),
        ],
    )(durations, max_len_arr)


def _sc_gather_body(table_hbm, idx_hbm, nvw_hbm, out_hbm,
                    idx_v, nvw_v, *rest):
    bufs = rest[:NBUF]
    gsems = rest[NBUF:2 * NBUF]
    ssems = rest[2 * NBUF:3 * NBUF]
    isem = rest[3 * NBUF]
    jsem = rest[3 * NBUF + 1]
    par = lax.axis_index("c")
    b = lax.axis_index("s")
    ibase = b * L + par * RPW
    PRE = NBUF - 1
    zvec = jnp.zeros((16,), jnp.float32)
    bufzero = [jnp.bool_(False)] * NBUF

    NV = D // 16

    def zero_rows(buf, lo, hi):
        def zero_vec(i, carry):
            buf[i // NV, pl.ds((i % NV) * 16, 16)] = zvec
            return carry
        lax.fori_loop(lo * NV, hi * NV, zero_vec, 0)

    def gchunk(c):
        return par * NCH + c

    def nval(c):
        return jnp.clip(vend_b - gchunk(c) * CH, 0, CH)

    def gather_copy(c):
        return pltpu.make_async_copy(
            table_hbm.at[idx_v.at[pl.ds(c * CH, CH)]],
            bufs[c % NBUF], gsems[c % NBUF])

    def start_gather(c):
        if c < PRE:
            gather_copy(c).start()
            return

        @pl.when(nval(c) > 0)
        def _():
            gather_copy(c).start()

    def wait_gather(c):
        if c < PRE:
            gather_copy(c).wait()
            return

        @pl.when(nval(c) > 0)
        def _():
            gather_copy(c).wait()

    def fix_tail(c):
        p = c % NBUF
        nv = nval(c)
        gathered = nv > 0 if c >= PRE else jnp.bool_(True)
        lo = jnp.where(bufzero[p] & jnp.logical_not(gathered), CH, nv)
        zero_rows(bufs[p], lo, CH)
        bufzero[p] = nv == 0

    def scatter_copy(c):
        return pltpu.make_async_copy(
            bufs[c % NBUF],
            out_hbm.at[pl.ds(b * L + gchunk(c) * CH, CH)], ssems[c % NBUF])

    nvw_copy = pltpu.make_async_copy(nvw_hbm, nvw_v, jsem)
    nvw_copy.start()
    idx_head = pltpu.make_async_copy(
        idx_hbm.at[pl.ds(ibase, PRE * CH)], idx_v.at[pl.ds(0, PRE * CH)], isem)
    idx_head.start()
    idx_head.wait()
    for c in range(PRE):
        start_gather(c)
    idx_tail = pltpu.make_async_copy(
        idx_hbm.at[pl.ds(ibase + PRE * CH, RPW - PRE * CH)],
        idx_v.at[pl.ds(PRE * CH, RPW - PRE * CH)], isem)
    idx_tail.start()
    nvw_copy.wait()
    vend_b = nvw_v[pl.ds(b, 16)][0]
    idx_tail.wait()

    for c in range(NCH):
        wait_gather(c)
        fix_tail(c)
        scatter_copy(c).start()
        nxt = c + NBUF - 1
        if nxt < NCH:
            if c >= 1:
                scatter_copy(c - 1).wait()
            start_gather(nxt)
    for cc in range(max(0, NCH - NBUF), NCH):
        scatter_copy(cc).wait()


@functools.cache
def _sc_gather():
    return pl.kernel(
        _sc_gather_body,
        mesh=plsc.VectorSubcoreMesh(core_axis_name="c", subcore_axis_name="s"),
        out_type=jax.ShapeDtypeStruct((B * L, D), jnp.float32),
        scratch_types=(
            [pltpu.VMEM((RPW,), jnp.int32), pltpu.VMEM((B + 16,), jnp.int32)]
            + [pltpu.VMEM((CH, D), jnp.float32)] * NBUF
            + [pltpu.SemaphoreType.DMA] * (2 * NBUF + 2)
        ),
    )


def kernel(x, durations, max_len):
    max_len_arr = jnp.asarray(max_len, jnp.int32).reshape(1)
    idx, mel_lens, nvw = _route(durations, max_len_arr)
    table = x.reshape(B * T, D)
    idx_flat = idx.reshape(B * L)
    out_flat = _sc_gather()(table, idx_flat, nvw)
    return out_flat.reshape(B, L, D), mel_lens


# speedup vs baseline: 1.0191x; 1.0191x over previous
import functools

import jax
import jax.numpy as jnp
from jax import lax
from jax.experimental import pallas as pl
from jax.experimental.pallas import tpu as pltpu
from jax.experimental.pallas import tpu_sc as plsc

B, T, D, L = 16, 512, 1024, 2048
NW = 32
RPW = (B * L) // NW
CH = 16
NCH = RPW // CH
NBUF = 6


def _route_body(dur_ref, maxlen_ref, idx_ref, mel_ref, nvw_ref):
    dur = jnp.maximum(dur_ref[...], 0)
    rk = lax.broadcasted_iota(jnp.int32, (T, T), 0)
    ci = lax.broadcasted_iota(jnp.int32, (T, T), 1)
    tri = (rk <= ci).astype(jnp.float32)
    cum = jnp.dot(dur.astype(jnp.float32), tri,
                  preferred_element_type=jnp.float32)
    cumi = cum.astype(jnp.int32)
    ones8 = jnp.ones((8, T), jnp.float32)
    pos = lax.broadcasted_iota(jnp.int32, (T, L), 1)
    for b in range(B):
        cmp = (cumi[b, :][:, None] <= pos).astype(jnp.float32)
        cnt = jnp.dot(ones8, cmp, preferred_element_type=jnp.float32)
        idx = jnp.minimum(cnt[0:1, :].astype(jnp.int32), T - 1)
        idx_ref[b, :, :] = b * T + idx
        total = jnp.sum(dur[b, :])
        mel_ref[b] = total
        nvw_ref[b] = jnp.clip(jnp.minimum(total, maxlen_ref[0]), 0, L)


def _route(durations, max_len_arr):
    return pl.pallas_call(
        _route_body,
        in_specs=[
            pl.BlockSpec((B, T), lambda: (0, 0)),
            pl.BlockSpec(memory_space=pltpu.SMEM),
        ],
        out_specs=[
            pl.BlockSpec((B, 1, L), lambda: (0, 0, 0)),
            pl.BlockSpec(memory_space=pltpu.SMEM),
            pl.BlockSpec(memory_space=pltpu.SMEM),
        ],
        out_shape=[
            jax.ShapeDtypeStruct((B, 1, L), jnp.int32),
            jax.ShapeDtypeStruct((B,), jnp.int32),
            jax.ShapeDtypeStruct((B + 16,), jnp.int32),
        ],
    )(durations, max_len_arr)


def _sc_gather_body(table_hbm, idx_hbm, nvw_hbm, out_hbm,
                    idx_v, nvw_v, *rest):
    bufs = rest[:NBUF]
    gsems = rest[NBUF:2 * NBUF]
    ssems = rest[2 * NBUF:3 * NBUF]
    isem = rest[3 * NBUF]
    jsem = rest[3 * NBUF + 1]
    par = lax.axis_index("c")
    b = lax.axis_index("s")
    ibase = b * L + par * RPW
    PRE = NBUF - 1
    zvec = jnp.zeros((16,), jnp.float32)
    bufzero = [jnp.bool_(False)] * NBUF

    NV = D // 16

    def zero_rows(buf, lo, hi):
        def zero_vec(i, carry):
            buf[i // NV, pl.ds((i % NV) * 16, 16)] = zvec
            return carry
        lax.fori_loop(lo * NV, hi * NV, zero_vec, 0)

    def gchunk(c):
        return par * NCH + c

    def nval(c):
        return jnp.clip(vend_b - gchunk(c) * CH, 0, CH)

    def gather_copy(c):
        return pltpu.make_async_copy(
            table_hbm.at[idx_v.at[pl.ds(c * CH, CH)]],
            bufs[c % NBUF], gsems[c % NBUF])

    def start_gather(c):
        if c < PRE:
            gather_copy(c).start()
            return

        @pl.when(nval(c) > 0)
        def _():
            gather_copy(c).start()

    def wait_gather(c):
        if c < PRE:
            gather_copy(c).wait()
            return

        @pl.when(nval(c) > 0)
        def _():
            gather_copy(c).wait()

    def fix_tail(c):
        p = c % NBUF
        nv = nval(c)
        gathered = nv > 0 if c >= PRE else jnp.bool_(True)
        lo = jnp.where(bufzero[p] & jnp.logical_not(gathered), CH, nv)
        zero_rows(bufs[p], lo, CH)
        bufzero[p] = nv == 0

    def scatter_copy(c):
        return pltpu.make_async_copy(
            bufs[c % NBUF],
            out_hbm.at[pl.ds(b * L + gchunk(c) * CH, CH)], ssems[c % NBUF])

    nvw_copy = pltpu.make_async_copy(nvw_hbm, nvw_v, jsem)
    nvw_copy.start()
    idx_head = pltpu.make_async_copy(
        idx_hbm.at[pl.ds(ibase, PRE * CH)], idx_v.at[pl.ds(0, PRE * CH)], isem)
    idx_head.start()
    idx_head.wait()
    for c in range(PRE):
        start_gather(c)
    idx_tail = pltpu.make_async_copy(
        idx_hbm.at[pl.ds(ibase + PRE * CH, RPW - PRE * CH)],
        idx_v.at[pl.ds(PRE * CH, RPW - PRE * CH)], isem)
    idx_tail.start()
    nvw_copy.wait()
    vend_b = nvw_v[pl.ds(b, 16)][0]
    idx_tail.wait()

    for c in range(NCH):
        wait_gather(c)
        fix_tail(c)
        scatter_copy(c).start()
        nxt = c + NBUF - 1
        if nxt < NCH:
            if c >= 1:
                scatter_copy(c - 1).wait()
            start_gather(nxt)
    for cc in range(max(0, NCH - NBUF), NCH):
        scatter_copy(cc).wait()


@functools.cache
def _sc_gather():
    return pl.kernel(
        _sc_gather_body,
        mesh=plsc.VectorSubcoreMesh(core_axis_name="c", subcore_axis_name="s"),
        out_type=jax.ShapeDtypeStruct((B * L, D), jnp.float32),
        scratch_types=(
            [pltpu.VMEM((RPW,), jnp.int32), pltpu.VMEM((B + 16,), jnp.int32)]
            + [pltpu.VMEM((CH, D), jnp.float32)] * NBUF
            + [pltpu.SemaphoreType.DMA] * (2 * NBUF + 2)
        ),
    )


def kernel(x, durations, max_len):
    max_len_arr = jnp.asarray(max_len, jnp.int32).reshape(1)
    idx, mel_lens, nvw = _route(durations, max_len_arr)
    table = x.reshape(B * T, D)
    idx_flat = idx.reshape(B * L)
    out_flat = _sc_gather()(table, idx_flat, nvw)
    return out_flat.reshape(B, L, D), mel_lens


# speedup vs baseline: 1.0991x; 1.0786x over previous
import functools

import jax
import jax.numpy as jnp
from jax import lax
from jax.experimental import pallas as pl
from jax.experimental.pallas import tpu as pltpu
from jax.experimental.pallas import tpu_sc as plsc

B, T, D, L = 16, 512, 1024, 2048
NW = 32
RPW = (B * L) // NW
CH = 16
NCH = RPW // CH
NBUF = 6


def _route_body(dur_ref, maxlen_ref, idx_ref, mel_ref, nvw_ref):
    dur = jnp.maximum(dur_ref[...], 0)
    rk = lax.broadcasted_iota(jnp.int32, (T, T), 0)
    ci = lax.broadcasted_iota(jnp.int32, (T, T), 1)
    tri = (rk <= ci).astype(jnp.float32)
    cum = jnp.dot(dur.astype(jnp.float32), tri,
                  preferred_element_type=jnp.float32)
    cumi = cum.astype(jnp.int32)
    ones8 = jnp.ones((8, T), jnp.float32)
    pos = lax.broadcasted_iota(jnp.int32, (T, L), 1)
    for b in range(B):
        cmp = (cumi[b, :][:, None] <= pos).astype(jnp.float32)
        cnt = jnp.dot(ones8, cmp, preferred_element_type=jnp.float32)
        idx = jnp.minimum(cnt[0:1, :].astype(jnp.int32), T - 1)
        idx_ref[b, :, :] = b * T + idx
        total = jnp.sum(dur[b, :])
        mel_ref[b] = total
        nvw_ref[b] = jnp.clip(jnp.minimum(total, maxlen_ref[0]), 0, L)


def _route(durations, max_len_arr):
    return pl.pallas_call(
        _route_body,
        in_specs=[
            pl.BlockSpec((B, T), lambda: (0, 0)),
            pl.BlockSpec(memory_space=pltpu.SMEM),
        ],
        out_specs=[
            pl.BlockSpec((B, 1, L), lambda: (0, 0, 0)),
            pl.BlockSpec(memory_space=pltpu.SMEM),
            pl.BlockSpec(memory_space=pltpu.SMEM),
        ],
        out_shape=[
            jax.ShapeDtypeStruct((B, 1, L), jnp.int32),
            jax.ShapeDtypeStruct((B,), jnp.int32),
            jax.ShapeDtypeStruct((B + 16,), jnp.int32),
        ],
    )(durations, max_len_arr)


def _sc_gather_body(table_hbm, idx_hbm, nvw_hbm, out_hbm,
                    idx_v, nvw_v, *rest):
    bufs = rest[:NBUF]
    gsems = rest[NBUF:2 * NBUF]
    ssems = rest[2 * NBUF:3 * NBUF]
    isem = rest[3 * NBUF]
    jsem = rest[3 * NBUF + 1]
    par = lax.axis_index("c")
    b = lax.axis_index("s")
    ibase = b * L + par * RPW
    PRE = NBUF - 1
    zvec = jnp.zeros((16,), jnp.float32)
    bufzero = [jnp.bool_(False)] * NBUF

    def zero_rows(buf, lo, hi):
        def zero_row(r, carry):
            for k in range(D // 16):
                buf[r, pl.ds(k * 16, 16)] = zvec
            return carry
        lax.fori_loop(lo, hi, zero_row, 0)

    def gchunk(c):
        return par * NCH + c

    def nval(c):
        return jnp.clip(vend_b - gchunk(c) * CH, 0, CH)

    def gather_copy(c):
        return pltpu.make_async_copy(
            table_hbm.at[idx_v.at[pl.ds(c * CH, CH)]],
            bufs[c % NBUF], gsems[c % NBUF])

    def start_gather(c):
        if c < PRE:
            gather_copy(c).start()
            return

        @pl.when(nval(c) > 0)
        def _():
            gather_copy(c).start()

    def wait_gather(c):
        if c < PRE:
            gather_copy(c).wait()
            return

        @pl.when(nval(c) > 0)
        def _():
            gather_copy(c).wait()

    def fix_tail(c):
        p = c % NBUF
        nv = nval(c)
        gathered = nv > 0 if c >= PRE else jnp.bool_(True)
        lo = jnp.where(bufzero[p] & jnp.logical_not(gathered), CH, nv)
        zero_rows(bufs[p], lo, CH)
        bufzero[p] = nv == 0

    def scatter_copy(c):
        return pltpu.make_async_copy(
            bufs[c % NBUF],
            out_hbm.at[pl.ds(b * L + gchunk(c) * CH, CH)], ssems[c % NBUF])

    nvw_copy = pltpu.make_async_copy(nvw_hbm, nvw_v, jsem)
    nvw_copy.start()
    idx_head = pltpu.make_async_copy(
        idx_hbm.at[pl.ds(ibase, PRE * CH)], idx_v.at[pl.ds(0, PRE * CH)], isem)
    idx_head.start()
    idx_head.wait()
    for c in range(PRE):
        start_gather(c)
    idx_tail = pltpu.make_async_copy(
        idx_hbm.at[pl.ds(ibase + PRE * CH, RPW - PRE * CH)],
        idx_v.at[pl.ds(PRE * CH, RPW - PRE * CH)], isem)
    idx_tail.start()
    nvw_copy.wait()
    vend_b = nvw_v[pl.ds(b, 16)][0]
    idx_tail.wait()

    for c in range(NCH):
        wait_gather(c)
        fix_tail(c)
        scatter_copy(c).start()
        nxt = c + NBUF - 1
        if nxt < NCH:
            if c >= 1:
                scatter_copy(c - 1).wait()
            start_gather(nxt)
    for cc in range(max(0, NCH - NBUF), NCH):
        scatter_copy(cc).wait()


@functools.cache
def _sc_gather():
    return pl.kernel(
        _sc_gather_body,
        mesh=plsc.VectorSubcoreMesh(core_axis_name="c", subcore_axis_name="s"),
        out_type=jax.ShapeDtypeStruct((B * L, D), jnp.float32),
        scratch_types=(
            [pltpu.VMEM((RPW,), jnp.int32), pltpu.VMEM((B + 16,), jnp.int32)]
            + [pltpu.VMEM((CH, D), jnp.float32)] * NBUF
            + [pltpu.SemaphoreType.DMA] * (2 * NBUF + 2)
        ),
    )


def kernel(x, durations, max_len):
    max_len_arr = jnp.asarray(max_len, jnp.int32).reshape(1)
    idx, mel_lens, nvw = _route(durations, max_len_arr)
    table = x.reshape(B * T, D)
    idx_flat = idx.reshape(B * L)
    out_flat = _sc_gather()(table, idx_flat, nvw)
    return out_flat.reshape(B, L, D), mel_lens


# speedup vs baseline: 1.1130x; 1.0126x over previous
import functools

import jax
import jax.numpy as jnp
from jax import lax
from jax.experimental import pallas as pl
from jax.experimental.pallas import tpu as pltpu
from jax.experimental.pallas import tpu_sc as plsc

B, T, D, L = 16, 512, 1024, 2048
NW = 32
RPW = (B * L) // NW
CH = 16
NCH = RPW // CH
NBUF = 6


def _route_body(dur_ref, maxlen_ref, idx_ref, mel_ref, nvw_ref):
    dur = jnp.maximum(dur_ref[...], 0)
    rk = lax.broadcasted_iota(jnp.int32, (T, T), 0)
    ci = lax.broadcasted_iota(jnp.int32, (T, T), 1)
    tri = (rk <= ci).astype(jnp.float32)
    cum = jnp.dot(dur.astype(jnp.float32), tri,
                  preferred_element_type=jnp.float32)
    cumi = cum.astype(jnp.int32)
    ones8 = jnp.ones((8, T), jnp.float32)
    pos = lax.broadcasted_iota(jnp.int32, (T, L), 1)
    for b in range(B):
        cmp = (cumi[b, :][:, None] <= pos).astype(jnp.float32)
        cnt = jnp.dot(ones8, cmp, preferred_element_type=jnp.float32)
        idx = jnp.minimum(cnt[0:1, :].astype(jnp.int32), T - 1)
        idx_ref[b, :, :] = b * T + idx
        total = jnp.sum(dur[b, :])
        mel_ref[b] = total
        nvw_ref[b] = jnp.clip(jnp.minimum(total, maxlen_ref[0]), 0, L)


def _route(durations, max_len_arr):
    return pl.pallas_call(
        _route_body,
        in_specs=[
            pl.BlockSpec((B, T), lambda: (0, 0)),
            pl.BlockSpec(memory_space=pltpu.SMEM),
        ],
        out_specs=[
            pl.BlockSpec((B, 1, L), lambda: (0, 0, 0)),
            pl.BlockSpec(memory_space=pltpu.SMEM),
            pl.BlockSpec(memory_space=pltpu.SMEM),
        ],
        out_shape=[
            jax.ShapeDtypeStruct((B, 1, L), jnp.int32),
            jax.ShapeDtypeStruct((B,), jnp.int32),
            jax.ShapeDtypeStruct((B + 16,), jnp.int32),
        ],
    )(durations, max_len_arr)


def _sc_gather_body(table_hbm, idx_hbm, nvw_hbm, out_hbm,
                    idx_v, nvw_v, *rest):
    bufs = rest[:NBUF]
    gsems = rest[NBUF:2 * NBUF]
    ssems = rest[2 * NBUF:3 * NBUF]
    isem = rest[3 * NBUF]
    jsem = rest[3 * NBUF + 1]
    par = lax.axis_index("c")
    b = lax.axis_index("s")
    ibase = b * L + par * RPW
    PRE = NBUF - 1
    zvec = jnp.zeros((16,), jnp.float32)
    bufzero = [jnp.bool_(False)] * NBUF

    def zero_rows(buf, lo, hi):
        def zero_row(r, carry):
            for k in range(D // 16):
                buf[r, pl.ds(k * 16, 16)] = zvec
            return carry
        lax.fori_loop(lo, hi, zero_row, 0)

    def gchunk(c):
        return par * NCH + c

    def nval(c):
        return jnp.clip(vend_b - gchunk(c) * CH, 0, CH)

    def gather_copy(c):
        return pltpu.make_async_copy(
            table_hbm.at[idx_v.at[pl.ds(c * CH, CH)]],
            bufs[c % NBUF], gsems[c % NBUF])

    def start_gather(c):
        if c < PRE:
            gather_copy(c).start()
            return

        @pl.when(nval(c) > 0)
        def _():
            gather_copy(c).start()

    def wait_gather(c):
        if c < PRE:
            gather_copy(c).wait()
            return

        @pl.when(nval(c) > 0)
        def _():
            gather_copy(c).wait()

    def fix_tail(c):
        p = c % NBUF
        nv = nval(c)
        gathered = nv > 0 if c >= PRE else jnp.bool_(True)
        lo = jnp.where(bufzero[p] & jnp.logical_not(gathered), CH, nv)
        zero_rows(bufs[p], lo, CH)
        bufzero[p] = nv == 0

    def scatter_copy(c):
        return pltpu.make_async_copy(
            bufs[c % NBUF],
            out_hbm.at[pl.ds(b * L + gchunk(c) * CH, CH)], ssems[c % NBUF])

    nvw_copy = pltpu.make_async_copy(nvw_hbm, nvw_v, jsem)
    nvw_copy.start()
    idx_head = pltpu.make_async_copy(
        idx_hbm.at[pl.ds(ibase, PRE * CH)], idx_v.at[pl.ds(0, PRE * CH)], isem)
    idx_head.start()
    idx_head.wait()
    for c in range(PRE):
        start_gather(c)
    idx_tail = pltpu.make_async_copy(
        idx_hbm.at[pl.ds(ibase + PRE * CH, RPW - PRE * CH)],
        idx_v.at[pl.ds(PRE * CH, RPW - PRE * CH)], isem)
    idx_tail.start()
    nvw_copy.wait()
    vend_b = nvw_v[pl.ds(b, 16)][0]
    idx_tail.wait()

    for c in range(NCH):
        nxt = c + NBUF - 1
        if nxt < NCH:
            if c >= 1:
                scatter_copy(c - 1).wait()
            start_gather(nxt)
        wait_gather(c)
        fix_tail(c)
        scatter_copy(c).start()
    for cc in range(max(0, NCH - NBUF), NCH):
        scatter_copy(cc).wait()


@functools.cache
def _sc_gather():
    return pl.kernel(
        _sc_gather_body,
        mesh=plsc.VectorSubcoreMesh(core_axis_name="c", subcore_axis_name="s"),
        out_type=jax.ShapeDtypeStruct((B * L, D), jnp.float32),
        scratch_types=(
            [pltpu.VMEM((RPW,), jnp.int32), pltpu.VMEM((B + 16,), jnp.int32)]
            + [pltpu.VMEM((CH, D), jnp.float32)] * NBUF
            + [pltpu.SemaphoreType.DMA] * (2 * NBUF + 2)
        ),
    )


def kernel(x, durations, max_len):
    max_len_arr = jnp.asarray(max_len, jnp.int32).reshape(1)
    idx, mel_lens, nvw = _route(durations, max_len_arr)
    table = x.reshape(B * T, D)
    idx_flat = idx.reshape(B * L)
    out_flat = _sc_gather()(table, idx_flat, nvw)
    return out_flat.reshape(B, L, D), mel_lens
